# baseline (speedup 1.0000x reference)
import functools

import jax
import jax.numpy as jnp
from jax import lax
from jax.experimental import pallas as pl
from jax.experimental.pallas import tpu as pltpu
from jax.experimental.pallas import tpu_sc as plsc

_B, _F, _D, _V = 16384, 26, 16, 1000000
_NC, _NS, _L = 2, 16, 16
_NW = _NC * _NS
_BPW = _B // _NW
_GC = 128
_NGC = _BPW // _GC
_CH = 64
_NCH = _BPW // _CH
_FD = _F * _D


def _fm_body(xt_hbm, femb_hbm, table_hbm, bias_hbm, out_hbm,
             idx_v, rows_v, dbuf_v, lr_v, bias_v, sem_g, sem_d):
    wid = lax.axis_index("s") * _NC + lax.axis_index("c")
    base = wid * _BPW

    pltpu.sync_copy(xt_hbm.at[:, pl.ds(base, _BPW)], idx_v)
    pltpu.sync_copy(bias_hbm, bias_v)

    pltpu.async_copy(
        femb_hbm.at[pl.ds(base, _CH), :], dbuf_v.at[pl.ds(0, _CH), :], sem_d)

    def gather_f(f, carry):
        cps = [
            pltpu.async_copy(
                table_hbm.at[idx_v.at[f, pl.ds(c * _GC, _GC)]],
                rows_v.at[f, pl.ds(c * _GC, _GC)],
                sem_g,
            )
            for c in range(_NGC)
        ]
        for cp in cps:
            cp.wait()
        return carry

    lax.fori_loop(0, _F, gather_f, 0)

    bias_vec = bias_v[...]

    def lr_g(g, carry):
        acc = bias_vec
        for f in range(_F):
            acc = acc + rows_v[f, pl.ds(g * _L, _L)]
        lr_v[pl.ds(g * _L, _L)] = acc
        return carry

    lax.fori_loop(0, _BPW // _L, lr_g, 0)

    lane = lax.iota(jnp.int32, _L)

    def chunk_t(t, carry):
        buf = t % 2
        pltpu.make_async_copy(
            femb_hbm.at[pl.ds(base, _CH), :],
            dbuf_v.at[pl.ds(0, _CH), :], sem_d).wait()

        @pl.when(t + 1 < _NCH)
        def _prefetch():
            pltpu.async_copy(
                femb_hbm.at[pl.ds(base + (t + 1) * _CH, _CH), :],
                dbuf_v.at[pl.ds(((t + 1) % 2) * _CH, _CH), :], sem_d)

        def grp(g, inner):
            e_idx = buf * _CH + g * _L + lane
            acc_sq = jnp.zeros((_L,), jnp.float32)
            acc_d = [jnp.zeros((_L,), jnp.float32) for _ in range(_D)]
            for f in range(_F):
                for d in range(_D):
                    fd = jnp.full((_L,), f * _D + d, jnp.int32)
                    v = plsc.load_gather(dbuf_v, [e_idx, fd])
                    acc_sq = acc_sq + v * v
                    acc_d[d] = acc_d[d] + v
            r = -acc_sq
            for d in range(_D):
                r = r + acc_d[d] * acc_d[d]
            gg = t * (_CH // _L) + g
            lr_v[pl.ds(gg * _L, _L)] = lr_v[pl.ds(gg * _L, _L)] + 0.5 * r
            return inner

        lax.fori_loop(0, _CH // _L, grp, 0)
        return carry

    lax.fori_loop(0, _NCH, chunk_t, 0)

    pltpu.sync_copy(lr_v, out_hbm.at[pl.ds(base, _BPW)])


@functools.cache
def _fm_sc():
    return functools.partial(
        pl.kernel,
        out_type=jax.ShapeDtypeStruct((_B,), jnp.float32),
        mesh=plsc.VectorSubcoreMesh(core_axis_name="c", subcore_axis_name="s"),
        scratch_types=[
            pltpu.VMEM((_F, _BPW), jnp.int32),
            pltpu.VMEM((_F, _BPW), jnp.float32),
            pltpu.VMEM((2 * _CH, _FD), jnp.float32),
            pltpu.VMEM((_BPW,), jnp.float32),
            pltpu.VMEM((_L,), jnp.float32),
            pltpu.SemaphoreType.DMA,
            pltpu.SemaphoreType.DMA,
        ],
        compiler_params=pltpu.CompilerParams(
            needs_layout_passes=False, use_tc_tiling_on_sc=False),
    )(_fm_body)


def kernel(X, feature_emb, lr_table, bias):
    Xt = jnp.asarray(X, jnp.int32).T
    femb = feature_emb.reshape(_B, _FD)
    table = lr_table.reshape(_V)
    bias16 = jnp.broadcast_to(bias.astype(jnp.float32), (_L,))
    out = _fm_sc()(Xt, femb, table, bias16)
    return out.reshape(_B, 1)


# speedup vs baseline: 1.4501x; 1.4501x over previous
import functools

import jax
import jax.numpy as jnp
from jax import lax
from jax.experimental import pallas as pl
from jax.experimental.pallas import tpu as pltpu
from jax.experimental.pallas import tpu_sc as plsc

_B, _F, _D, _V = 16384, 26, 16, 1000000
_NC, _NS, _L = 2, 16, 16
_NW = _NC * _NS
_BPW = _B // _NW
_GC = 128
_NGC = _BPW // _GC
_CH = 64
_NCH = _BPW // _CH
_FD = _F * _D
_FDP = _FD + 1


def _fm_body(xt_hbm, femb_hbm, table_hbm, bias_hbm, out_hbm,
             idx_v, rows_v, dbuf_v, lr_v, bias_v, sem_g, sem_d):
    wid = lax.axis_index("s") * _NC + lax.axis_index("c")
    base = wid * _BPW

    pltpu.sync_copy(xt_hbm.at[:, pl.ds(base, _BPW)], idx_v)
    pltpu.sync_copy(bias_hbm, bias_v)

    pltpu.async_copy(
        femb_hbm.at[pl.ds(base, _CH), :],
        dbuf_v.at[pl.ds(0, _CH), pl.ds(0, _FD)], sem_d)

    def gather_f(f, carry):
        for c in range(_NGC):
            pltpu.async_copy(
                table_hbm.at[idx_v.at[f, pl.ds(c * _GC, _GC)]],
                rows_v.at[f, pl.ds(c * _GC, _GC)],
                sem_g,
            )
        return carry

    def drain_f(f, carry):
        for c in range(_NGC):
            pltpu.make_async_copy(
                table_hbm.at[idx_v.at[f, pl.ds(c * _GC, _GC)]],
                rows_v.at[f, pl.ds(c * _GC, _GC)],
                sem_g,
            ).wait()
        return carry

    lax.fori_loop(0, _F, gather_f, 0)
    lax.fori_loop(0, _F, drain_f, 0)

    bias_vec = bias_v[...]

    def lr_g(g, carry):
        acc = bias_vec
        for f in range(_F):
            acc = acc + rows_v[f, pl.ds(g * _L, _L)]
        lr_v[pl.ds(g * _L, _L)] = acc
        return carry

    lax.fori_loop(0, _BPW // _L, lr_g, 0)

    lane = lax.iota(jnp.int32, _L)

    def chunk_t(t, carry):
        buf = t % 2
        pltpu.make_async_copy(
            femb_hbm.at[pl.ds(base, _CH), :],
            dbuf_v.at[pl.ds(0, _CH), pl.ds(0, _FD)], sem_d).wait()

        @pl.when(t + 1 < _NCH)
        def _prefetch():
            pltpu.async_copy(
                femb_hbm.at[pl.ds(base + (t + 1) * _CH, _CH), :],
                dbuf_v.at[pl.ds(((t + 1) % 2) * _CH, _CH), pl.ds(0, _FD)],
                sem_d)

        def grp(g, inner):
            e_idx = buf * _CH + g * _L + lane
            acc_sq = jnp.zeros((_L,), jnp.float32)
            acc_d = [jnp.zeros((_L,), jnp.float32) for _ in range(_D)]
            for f in range(_F):
                for d in range(_D):
                    fd = jnp.full((_L,), f * _D + d, jnp.int32)
                    v = plsc.load_gather(dbuf_v, [e_idx, fd])
                    acc_sq = acc_sq + v * v
                    acc_d[d] = acc_d[d] + v
            r = -acc_sq
            for d in range(_D):
                r = r + acc_d[d] * acc_d[d]
            gg = t * (_CH // _L) + g
            lr_v[pl.ds(gg * _L, _L)] = lr_v[pl.ds(gg * _L, _L)] + 0.5 * r
            return inner

        lax.fori_loop(0, _CH // _L, grp, 0)
        return carry

    lax.fori_loop(0, _NCH, chunk_t, 0)

    pltpu.sync_copy(lr_v, out_hbm.at[pl.ds(base, _BPW)])


@functools.cache
def _fm_sc():
    return functools.partial(
        pl.kernel,
        out_type=jax.ShapeDtypeStruct((_B,), jnp.float32),
        mesh=plsc.VectorSubcoreMesh(core_axis_name="c", subcore_axis_name="s"),
        scratch_types=[
            pltpu.VMEM((_F, _BPW), jnp.int32),
            pltpu.VMEM((_F, _BPW), jnp.float32),
            pltpu.VMEM((2 * _CH, _FDP), jnp.float32),
            pltpu.VMEM((_BPW,), jnp.float32),
            pltpu.VMEM((_L,), jnp.float32),
            pltpu.SemaphoreType.DMA,
            pltpu.SemaphoreType.DMA,
        ],
        compiler_params=pltpu.CompilerParams(
            needs_layout_passes=False, use_tc_tiling_on_sc=False),
    )(_fm_body)


def kernel(X, feature_emb, lr_table, bias):
    Xt = jnp.asarray(X, jnp.int32).T
    femb = feature_emb.reshape(_B, _FD)
    table = lr_table.reshape(_V)
    bias16 = jnp.broadcast_to(bias.astype(jnp.float32), (_L,))
    out = _fm_sc()(Xt, femb, table, bias16)
    return out.reshape(_B, 1)


# speedup vs baseline: 2.2614x; 1.5595x over previous
import functools

import jax
import jax.numpy as jnp
from jax import lax
from jax.experimental import pallas as pl
from jax.experimental.pallas import tpu as pltpu
from jax.experimental.pallas import tpu_sc as plsc

_B, _F, _D, _V = 16384, 26, 16, 1000000
_NC, _NS, _L = 2, 16, 16
_NW = _NC * _NS
_BPW = _B // _NW
_GC = 128
_NGC = _BPW // _GC
_CH = 64
_NCH = _BPW // _CH
_FD = _F * _D


def _fm_body(xt_hbm, fembt_hbm, table_hbm, bias_hbm, out_hbm,
             idx_v, rows_v, dbuf_v, lr_v, bias_v, sem_g, sem_d):
    wid = lax.axis_index("s") * _NC + lax.axis_index("c")
    base = wid * _BPW

    pltpu.sync_copy(xt_hbm.at[:, pl.ds(base, _BPW)], idx_v)
    pltpu.sync_copy(bias_hbm, bias_v)

    pltpu.async_copy(
        fembt_hbm.at[:, pl.ds(base, _CH)],
        dbuf_v.at[:, pl.ds(0, _CH)], sem_d)

    def gather_f(f, carry):
        for c in range(_NGC):
            pltpu.async_copy(
                table_hbm.at[0].at[idx_v.at[f, pl.ds(c * _GC, _GC)]],
                rows_v.at[f, pl.ds(c * _GC, _GC)],
                sem_g,
            )
        return carry

    def drain_f(f, carry):
        for c in range(_NGC):
            pltpu.make_async_copy(
                table_hbm.at[0].at[idx_v.at[f, pl.ds(c * _GC, _GC)]],
                rows_v.at[f, pl.ds(c * _GC, _GC)],
                sem_g,
            ).wait()
        return carry

    lax.fori_loop(0, _F, gather_f, 0)
    lax.fori_loop(0, _F, drain_f, 0)

    bias_vec = bias_v[...]
    lane = lax.iota(jnp.int32, _L)

    def lr_g(g, carry):
        acc = bias_vec
        for f in range(_F):
            acc = acc + rows_v[f, pl.ds(g * _L, _L)]
        lr_v[pl.ds(g * _L, _L)] = acc
        return carry

    lax.fori_loop(0, _BPW // _L, lr_g, 0)

    def chunk_t(t, carry):
        buf = t % 2
        pltpu.make_async_copy(
            fembt_hbm.at[:, pl.ds(base, _CH)],
            dbuf_v.at[:, pl.ds(0, _CH)], sem_d).wait()

        @pl.when(t + 1 < _NCH)
        def _prefetch():
            pltpu.async_copy(
                fembt_hbm.at[:, pl.ds(base + (t + 1) * _CH, _CH)],
                dbuf_v.at[:, pl.ds(((t + 1) % 2) * _CH, _CH)],
                sem_d)

        def grp(g, inner):
            off = buf * _CH + g * _L
            acc_sq = jnp.zeros((_L,), jnp.float32)
            acc_d = [jnp.zeros((_L,), jnp.float32) for _ in range(_D)]
            for f in range(_F):
                for d in range(_D):
                    v = dbuf_v[f * _D + d, pl.ds(off, _L)]
                    acc_sq = acc_sq + v * v
                    acc_d[d] = acc_d[d] + v
            r = -acc_sq
            for d in range(_D):
                r = r + acc_d[d] * acc_d[d]
            gg = t * (_CH // _L) + g
            lr_v[pl.ds(gg * _L, _L)] = lr_v[pl.ds(gg * _L, _L)] + 0.5 * r
            return inner

        lax.fori_loop(0, _CH // _L, grp, 0)
        return carry

    lax.fori_loop(0, _NCH, chunk_t, 0)

    pltpu.sync_copy(lr_v, out_hbm.at[pl.ds(base, _BPW)])


@functools.cache
def _fm_sc():
    return functools.partial(
        pl.kernel,
        out_type=jax.ShapeDtypeStruct((_B,), jnp.float32),
        mesh=plsc.VectorSubcoreMesh(core_axis_name="c", subcore_axis_name="s"),
        scratch_types=[
            pltpu.VMEM((_F, _BPW), jnp.int32),
            pltpu.VMEM((_F, _BPW), jnp.float32),
            pltpu.VMEM((_FD, 2 * _CH), jnp.float32),
            pltpu.VMEM((_BPW,), jnp.float32),
            pltpu.VMEM((_L,), jnp.float32),
            pltpu.SemaphoreType.DMA,
            pltpu.SemaphoreType.DMA,
        ],
        compiler_params=pltpu.CompilerParams(
            needs_layout_passes=False, use_tc_tiling_on_sc=False),
    )(_fm_body)


def kernel(X, feature_emb, lr_table, bias):
    Xt = jnp.asarray(X, jnp.int32).T
    fembt = feature_emb.reshape(_B, _FD).T
    bias16 = jnp.broadcast_to(bias.astype(jnp.float32), (_L,))
    out = _fm_sc()(Xt, fembt, lr_table.T, bias16)
    return out.reshape(_B, 1)


# speedup vs baseline: 2.4446x; 1.0810x over previous
import functools

import jax
import jax.numpy as jnp
from jax import lax
from jax.experimental import pallas as pl
from jax.experimental.pallas import tpu as pltpu
from jax.experimental.pallas import tpu_sc as plsc

_B, _F, _D, _V = 16384, 26, 16, 1000000
_NC, _NS, _L = 2, 16, 16
_NW = _NC * _NS
_BPW = _B // _NW
_GC = 128
_NGC = _BPW // _GC
_CH = 128
_NCH = _BPW // _CH
_FD = _F * _D


def _dot_body(fembt_hbm, out_hbm, dbuf_v, dot_v, sem_d):
    wid = lax.axis_index("s") * _NC + lax.axis_index("c")
    base = wid * _BPW

    pltpu.async_copy(
        fembt_hbm.at[:, pl.ds(base, _CH)],
        dbuf_v.at[:, pl.ds(0, _CH)], sem_d)

    def chunk_t(t, carry):
        buf = t % 2
        pltpu.make_async_copy(
            fembt_hbm.at[:, pl.ds(base, _CH)],
            dbuf_v.at[:, pl.ds(0, _CH)], sem_d).wait()

        @pl.when(t + 1 < _NCH)
        def _prefetch():
            pltpu.async_copy(
                fembt_hbm.at[:, pl.ds(base + (t + 1) * _CH, _CH)],
                dbuf_v.at[:, pl.ds(((t + 1) % 2) * _CH, _CH)],
                sem_d)

        def grp(g, inner):
            off = buf * _CH + g * _L
            acc_sq = jnp.zeros((_L,), jnp.float32)
            acc_d = [jnp.zeros((_L,), jnp.float32) for _ in range(_D)]
            for f in range(_F):
                for d in range(_D):
                    v = dbuf_v[f * _D + d, pl.ds(off, _L)]
                    acc_sq = acc_sq + v * v
                    acc_d[d] = acc_d[d] + v
            r = -acc_sq
            for d in range(_D):
                r = r + acc_d[d] * acc_d[d]
            gg = t * (_CH // _L) + g
            dot_v[pl.ds(gg * _L, _L)] = 0.5 * r
            return inner

        lax.fori_loop(0, _CH // _L, grp, 0)
        return carry

    lax.fori_loop(0, _NCH, chunk_t, 0)

    pltpu.sync_copy(dot_v, out_hbm.at[pl.ds(base, _BPW)])


def _lr_body(xt_hbm, table_hbm, bias_hbm, dot_hbm, out_hbm,
             idx_v, rows_v, lr_v, bias_v, sem_g, sem_d):
    wid = lax.axis_index("s") * _NC + lax.axis_index("c")
    base = wid * _BPW

    pltpu.sync_copy(xt_hbm.at[:, pl.ds(base, _BPW)], idx_v)
    pltpu.async_copy(dot_hbm.at[pl.ds(base, _BPW)], lr_v, sem_d)
    pltpu.sync_copy(bias_hbm, bias_v)

    def gather_f(f, carry):
        for c in range(_NGC):
            pltpu.async_copy(
                table_hbm.at[0].at[idx_v.at[f, pl.ds(c * _GC, _GC)]],
                rows_v.at[f, pl.ds(c * _GC, _GC)],
                sem_g,
            )
        return carry

    def drain_f(f, carry):
        for c in range(_NGC):
            pltpu.make_async_copy(
                table_hbm.at[0].at[idx_v.at[f, pl.ds(c * _GC, _GC)]],
                rows_v.at[f, pl.ds(c * _GC, _GC)],
                sem_g,
            ).wait()
        return carry

    lax.fori_loop(0, _F, gather_f, 0)
    pltpu.make_async_copy(
        dot_hbm.at[pl.ds(base, _BPW)], lr_v, sem_d).wait()
    lax.fori_loop(0, _F, drain_f, 0)

    bias_vec = bias_v[...]

    def lr_g(g, carry):
        acc = lr_v[pl.ds(g * _L, _L)] + bias_vec
        for f in range(_F):
            acc = acc + rows_v[f, pl.ds(g * _L, _L)]
        lr_v[pl.ds(g * _L, _L)] = acc
        return carry

    lax.fori_loop(0, _BPW // _L, lr_g, 0)

    pltpu.sync_copy(lr_v, out_hbm.at[pl.ds(base, _BPW)])


def _params():
    return pltpu.CompilerParams(
        needs_layout_passes=False, use_tc_tiling_on_sc=False)


@functools.cache
def _dot_sc():
    return functools.partial(
        pl.kernel,
        out_type=jax.ShapeDtypeStruct((_B,), jnp.float32),
        mesh=plsc.VectorSubcoreMesh(core_axis_name="c", subcore_axis_name="s"),
        scratch_types=[
            pltpu.VMEM((_FD, 2 * _CH), jnp.float32),
            pltpu.VMEM((_BPW,), jnp.float32),
            pltpu.SemaphoreType.DMA,
        ],
        compiler_params=_params(),
    )(_dot_body)


@functools.cache
def _lr_sc():
    return functools.partial(
        pl.kernel,
        out_type=jax.ShapeDtypeStruct((_B,), jnp.float32),
        mesh=plsc.VectorSubcoreMesh(core_axis_name="c", subcore_axis_name="s"),
        scratch_types=[
            pltpu.VMEM((_F, _BPW), jnp.int32),
            pltpu.VMEM((_F, _BPW), jnp.float32),
            pltpu.VMEM((_BPW,), jnp.float32),
            pltpu.VMEM((_L,), jnp.float32),
            pltpu.SemaphoreType.DMA,
            pltpu.SemaphoreType.DMA,
        ],
        compiler_params=_params(),
    )(_lr_body)


def kernel(X, feature_emb, lr_table, bias):
    Xt = jnp.asarray(X, jnp.int32).T
    fembt = feature_emb.reshape(_B, _FD).T
    bias16 = jnp.broadcast_to(bias.astype(jnp.float32), (_L,))
    dot = _dot_sc()(fembt)
    out = _lr_sc()(Xt, lr_table.T, bias16, dot)
    return out.reshape(_B, 1)


# speedup vs baseline: 3.1691x; 1.2964x over previous
import functools

import jax
import jax.numpy as jnp
from jax import lax
from jax.experimental import pallas as pl
from jax.experimental.pallas import tpu as pltpu
from jax.experimental.pallas import tpu_sc as plsc

_B, _F, _D, _V = 16384, 26, 16, 1000000
_NC, _NS, _L = 2, 16, 16
_NW = _NC * _NS
_BPW = _B // _NW
_GC = 128
_NGC = _BPW // _GC
_CH = 128
_NCH = _BPW // _CH
_FD = _F * _D


def _dot_body(fembt_hbm, out_hbm, dbuf_v, dot_v, sem_d):
    wid = lax.axis_index("s") * _NC + lax.axis_index("c")
    base = wid * _BPW

    pltpu.async_copy(
        fembt_hbm.at[:, pl.ds(base, _CH)],
        dbuf_v.at[:, pl.ds(0, _CH)], sem_d)

    def chunk_t(t, carry):
        buf = t % 2
        pltpu.make_async_copy(
            fembt_hbm.at[:, pl.ds(base, _CH)],
            dbuf_v.at[:, pl.ds(0, _CH)], sem_d).wait()

        @pl.when(t + 1 < _NCH)
        def _prefetch():
            pltpu.async_copy(
                fembt_hbm.at[:, pl.ds(base + (t + 1) * _CH, _CH)],
                dbuf_v.at[:, pl.ds(((t + 1) % 2) * _CH, _CH)],
                sem_d)

        def grp(g, inner):
            off = buf * _CH + g * _L
            acc_sq = jnp.zeros((_L,), jnp.float32)
            acc_d = [jnp.zeros((_L,), jnp.float32) for _ in range(_D)]
            for f in range(_F):
                for d in range(_D):
                    v = dbuf_v[f * _D + d, pl.ds(off, _L)]
                    acc_sq = acc_sq + v * v
                    acc_d[d] = acc_d[d] + v
            r = -acc_sq
            for d in range(_D):
                r = r + acc_d[d] * acc_d[d]
            gg = t * (_CH // _L) + g
            dot_v[pl.ds(gg * _L, _L)] = 0.5 * r
            return inner

        lax.fori_loop(0, _CH // _L, grp, 0)
        return carry

    lax.fori_loop(0, _NCH, chunk_t, 0)

    pltpu.sync_copy(dot_v, out_hbm.at[pl.ds(base, _BPW)])


def _lr_body(xt_hbm, table_hbm, bias_hbm, dot_hbm, out_hbm,
             idx_v, rows_v, lr_v, bias_v, sem_g, sem_d):
    wid = lax.axis_index("s") * _NC + lax.axis_index("c")
    base = wid * _BPW

    pltpu.sync_copy(xt_hbm.at[:, pl.ds(base, _BPW)], idx_v)
    pltpu.async_copy(dot_hbm.at[pl.ds(base, _BPW)], lr_v, sem_d)
    pltpu.sync_copy(bias_hbm, bias_v)

    def gather_f(f, carry):
        for c in range(_NGC):
            pltpu.async_copy(
                table_hbm.at[0].at[idx_v.at[f, pl.ds(c * _GC, _GC)]],
                rows_v.at[f, pl.ds(c * _GC, _GC)],
                sem_g,
            )
        return carry

    def drain_f(f, carry):
        for c in range(_NGC):
            pltpu.make_async_copy(
                table_hbm.at[0].at[idx_v.at[f, pl.ds(c * _GC, _GC)]],
                rows_v.at[f, pl.ds(c * _GC, _GC)],
                sem_g,
            ).wait()
        return carry

    lax.fori_loop(0, _F, gather_f, 0)
    pltpu.make_async_copy(
        dot_hbm.at[pl.ds(base, _BPW)], lr_v, sem_d).wait()
    lax.fori_loop(0, _F, drain_f, 0)

    bias_vec = bias_v[...]

    def lr_g(g, carry):
        acc = lr_v[pl.ds(g * _L, _L)] + bias_vec
        for f in range(_F):
            acc = acc + rows_v[f, pl.ds(g * _L, _L)]
        lr_v[pl.ds(g * _L, _L)] = acc
        return carry

    lax.fori_loop(0, _BPW // _L, lr_g, 0)

    pltpu.sync_copy(lr_v, out_hbm.at[pl.ds(base, _BPW)])


def _params(tc_tiling=False):
    return pltpu.CompilerParams(
        needs_layout_passes=False, use_tc_tiling_on_sc=tc_tiling)


@functools.cache
def _dot_sc():
    return functools.partial(
        pl.kernel,
        out_type=jax.ShapeDtypeStruct((_B,), jnp.float32),
        mesh=plsc.VectorSubcoreMesh(core_axis_name="c", subcore_axis_name="s"),
        scratch_types=[
            pltpu.VMEM((_FD, 2 * _CH), jnp.float32),
            pltpu.VMEM((_BPW,), jnp.float32),
            pltpu.SemaphoreType.DMA,
        ],
        compiler_params=_params(tc_tiling=True),
    )(_dot_body)


@functools.cache
def _lr_sc():
    return functools.partial(
        pl.kernel,
        out_type=jax.ShapeDtypeStruct((_B,), jnp.float32),
        mesh=plsc.VectorSubcoreMesh(core_axis_name="c", subcore_axis_name="s"),
        scratch_types=[
            pltpu.VMEM((_F, _BPW), jnp.int32),
            pltpu.VMEM((_F, _BPW), jnp.float32),
            pltpu.VMEM((_BPW,), jnp.float32),
            pltpu.VMEM((_L,), jnp.float32),
            pltpu.SemaphoreType.DMA,
            pltpu.SemaphoreType.DMA,
        ],
        compiler_params=_params(),
    )(_lr_body)


def kernel(X, feature_emb, lr_table, bias):
    Xt = jnp.asarray(X, jnp.int32).T
    fembt = feature_emb.reshape(_B, _FD).T
    bias16 = jnp.broadcast_to(bias.astype(jnp.float32), (_L,))
    dot = _dot_sc()(fembt)
    out = _lr_sc()(Xt, lr_table.T, bias16, dot)
    return out.reshape(_B, 1)


# speedup vs baseline: 3.5234x; 1.1118x over previous
import functools

import jax
import jax.numpy as jnp
from jax import lax
from jax.experimental import pallas as pl
from jax.experimental.pallas import tpu as pltpu
from jax.experimental.pallas import tpu_sc as plsc

_B, _F, _D, _V = 16384, 26, 16, 1000000
_NC, _NS, _L = 2, 16, 16
_NW = _NC * _NS
_BPW = _B // _NW
_GC = 128
_NGC = _BPW // _GC
_CH = 128
_NCH = _BPW // _CH
_FD = _F * _D


def _dot_body(fembt_hbm, out_hbm, dbuf_v, dot_v, sem_d):
    wid = lax.axis_index("s") * _NC + lax.axis_index("c")
    base = wid * _BPW

    pltpu.async_copy(
        fembt_hbm.at[:, pl.ds(base, _CH)],
        dbuf_v.at[:, pl.ds(0, _CH)], sem_d)

    def chunk_t(t, carry):
        buf = t % 2
        pltpu.make_async_copy(
            fembt_hbm.at[:, pl.ds(base, _CH)],
            dbuf_v.at[:, pl.ds(0, _CH)], sem_d).wait()

        @pl.when(t + 1 < _NCH)
        def _prefetch():
            pltpu.async_copy(
                fembt_hbm.at[:, pl.ds(base + (t + 1) * _CH, _CH)],
                dbuf_v.at[:, pl.ds(((t + 1) % 2) * _CH, _CH)],
                sem_d)

        def grp(g, inner):
            off = buf * _CH + g * _L
            acc_sq = jnp.zeros((_L,), jnp.float32)
            acc_d = [jnp.zeros((_L,), jnp.float32) for _ in range(_D)]
            for f in range(_F):
                for d in range(_D):
                    v = dbuf_v[f * _D + d, pl.ds(off, _L)]
                    acc_sq = acc_sq + v * v
                    acc_d[d] = acc_d[d] + v
            r = -acc_sq
            for d in range(_D):
                r = r + acc_d[d] * acc_d[d]
            gg = t * (_CH // _L) + g
            dot_v[pl.ds(gg * _L, _L)] = 0.5 * r
            return inner

        lax.fori_loop(0, _CH // _L, grp, 0)
        return carry

    lax.fori_loop(0, _NCH, chunk_t, 0)

    pltpu.sync_copy(dot_v, out_hbm.at[pl.ds(base, _BPW)])


_SC_CH = 4096
_SC_FULL = _V // _SC_CH
_SC_REM = _V - _SC_FULL * _SC_CH


def _lr_body(xt_hbm, table_hbm, bias_hbm, dot_hbm, out_hbm,
             idx_v, rows_v, lr_v, bias_v, shared_v, sem_g, sem_d, sem_t):
    wid = lax.axis_index("s") * _NC + lax.axis_index("c")
    sid = lax.axis_index("s")
    base = wid * _BPW

    def stage_k(k, carry):
        cid = k * _NS + sid

        @pl.when(cid < _SC_FULL)
        def _full():
            start = cid * _SC_CH
            pltpu.async_copy(table_hbm.at[0, pl.ds(start, _SC_CH)],
                             shared_v.at[pl.ds(start, _SC_CH)], sem_t)

        @pl.when(cid == _SC_FULL)
        def _partial():
            start = cid * _SC_CH
            pltpu.async_copy(table_hbm.at[0, pl.ds(start, _SC_REM)],
                             shared_v.at[pl.ds(start, _SC_REM)], sem_t)

        return carry

    _N_STAGE = (_SC_FULL + _SC_REM // _SC_CH) // _NS + 2
    lax.fori_loop(0, _N_STAGE, stage_k, 0)

    pltpu.sync_copy(xt_hbm.at[:, pl.ds(base, _BPW)], idx_v)
    pltpu.async_copy(dot_hbm.at[pl.ds(base, _BPW)], lr_v, sem_d)
    pltpu.sync_copy(bias_hbm, bias_v)

    def stage_drain(k, carry):
        cid = k * _NS + sid

        @pl.when(cid < _SC_FULL)
        def _full():
            start = cid * _SC_CH
            pltpu.make_async_copy(table_hbm.at[0, pl.ds(start, _SC_CH)],
                                  shared_v.at[pl.ds(start, _SC_CH)], sem_t).wait()

        @pl.when(cid == _SC_FULL)
        def _partial():
            start = cid * _SC_CH
            pltpu.make_async_copy(table_hbm.at[0, pl.ds(start, _SC_REM)],
                                  shared_v.at[pl.ds(start, _SC_REM)], sem_t).wait()

        return carry

    lax.fori_loop(0, _N_STAGE, stage_drain, 0)
    plsc.subcore_barrier()

    def gather_f(f, carry):
        for c in range(_NGC):
            pltpu.async_copy(
                shared_v.at[idx_v.at[f, pl.ds(c * _GC, _GC)]],
                rows_v.at[f, pl.ds(c * _GC, _GC)],
                sem_g,
            )
        return carry

    def drain_f(f, carry):
        for c in range(_NGC):
            pltpu.make_async_copy(
                shared_v.at[idx_v.at[f, pl.ds(c * _GC, _GC)]],
                rows_v.at[f, pl.ds(c * _GC, _GC)],
                sem_g,
            ).wait()
        return carry

    lax.fori_loop(0, _F, gather_f, 0)
    pltpu.make_async_copy(
        dot_hbm.at[pl.ds(base, _BPW)], lr_v, sem_d).wait()
    lax.fori_loop(0, _F, drain_f, 0)

    bias_vec = bias_v[...]

    def lr_g(g, carry):
        acc = lr_v[pl.ds(g * _L, _L)] + bias_vec
        for f in range(_F):
            acc = acc + rows_v[f, pl.ds(g * _L, _L)]
        lr_v[pl.ds(g * _L, _L)] = acc
        return carry

    lax.fori_loop(0, _BPW // _L, lr_g, 0)

    pltpu.sync_copy(lr_v, out_hbm.at[pl.ds(base, _BPW)])


def _params(tc_tiling=False):
    return pltpu.CompilerParams(
        needs_layout_passes=False, use_tc_tiling_on_sc=tc_tiling)


@functools.cache
def _dot_sc():
    return functools.partial(
        pl.kernel,
        out_type=jax.ShapeDtypeStruct((_B,), jnp.float32),
        mesh=plsc.VectorSubcoreMesh(core_axis_name="c", subcore_axis_name="s"),
        scratch_types=[
            pltpu.VMEM((_FD, 2 * _CH), jnp.float32),
            pltpu.VMEM((_BPW,), jnp.float32),
            pltpu.SemaphoreType.DMA,
        ],
        compiler_params=_params(tc_tiling=True),
    )(_dot_body)


@functools.cache
def _lr_sc():
    return functools.partial(
        pl.kernel,
        out_type=jax.ShapeDtypeStruct((_B,), jnp.float32),
        mesh=plsc.VectorSubcoreMesh(core_axis_name="c", subcore_axis_name="s"),
        scratch_types=[
            pltpu.VMEM((_F, _BPW), jnp.int32),
            pltpu.VMEM((_F, _BPW), jnp.float32),
            pltpu.VMEM((_BPW,), jnp.float32),
            pltpu.VMEM((_L,), jnp.float32),
            pltpu.VMEM_SHARED((_V,), jnp.float32),
            pltpu.SemaphoreType.DMA,
            pltpu.SemaphoreType.DMA,
            pltpu.SemaphoreType.DMA,
        ],
        compiler_params=_params(),
    )(_lr_body)


def kernel(X, feature_emb, lr_table, bias):
    Xt = jnp.asarray(X, jnp.int32).T
    fembt = feature_emb.reshape(_B, _FD).T
    bias16 = jnp.broadcast_to(bias.astype(jnp.float32), (_L,))
    dot = _dot_sc()(fembt)
    out = _lr_sc()(Xt, lr_table.T, bias16, dot)
    return out.reshape(_B, 1)


# speedup vs baseline: 3.5339x; 1.0030x over previous
import functools

import jax
import jax.numpy as jnp
from jax import lax
from jax.experimental import pallas as pl
from jax.experimental.pallas import tpu as pltpu
from jax.experimental.pallas import tpu_sc as plsc

_B, _F, _D, _V = 16384, 26, 16, 1000000
_NC, _NS, _L = 2, 16, 16
_NW = _NC * _NS
_BPW = _B // _NW
_GC = 128
_NGC = _BPW // _GC
_CH = 128
_NCH = _BPW // _CH
_FD = _F * _D


def _dot_body(fembt_hbm, out_hbm, dbuf_v, dot_v, sem_d):
    wid = lax.axis_index("s") * _NC + lax.axis_index("c")
    base = wid * _BPW

    pltpu.async_copy(
        fembt_hbm.at[:, pl.ds(base, _CH)],
        dbuf_v.at[:, pl.ds(0, _CH)], sem_d)

    def chunk_t(t, carry):
        buf = t % 2
        pltpu.make_async_copy(
            fembt_hbm.at[:, pl.ds(base, _CH)],
            dbuf_v.at[:, pl.ds(0, _CH)], sem_d).wait()

        @pl.when(t + 1 < _NCH)
        def _prefetch():
            pltpu.async_copy(
                fembt_hbm.at[:, pl.ds(base + (t + 1) * _CH, _CH)],
                dbuf_v.at[:, pl.ds(((t + 1) % 2) * _CH, _CH)],
                sem_d)

        def grp(g, inner):
            off = buf * _CH + g * _L
            zero = jnp.zeros((_L,), jnp.float32)

            def facc(f, carry):
                acc_sq = carry[0]
                acc_d = list(carry[1:])
                for d in range(_D):
                    v = dbuf_v[f * _D + d, pl.ds(off, _L)]
                    acc_sq = acc_sq + v * v
                    acc_d[d] = acc_d[d] + v
                return (acc_sq, *acc_d)

            res = lax.fori_loop(0, _F, facc, (zero,) * (_D + 1))
            r = -res[0]
            for d in range(_D):
                r = r + res[1 + d] * res[1 + d]
            gg = t * (_CH // _L) + g
            dot_v[pl.ds(gg * _L, _L)] = 0.5 * r
            return inner

        lax.fori_loop(0, _CH // _L, grp, 0)
        return carry

    lax.fori_loop(0, _NCH, chunk_t, 0)

    pltpu.sync_copy(dot_v, out_hbm.at[pl.ds(base, _BPW)])


_SC_CH = 4096
_SC_FULL = _V // _SC_CH
_SC_REM = _V - _SC_FULL * _SC_CH


def _lr_body(xt_hbm, table_hbm, bias_hbm, dot_hbm, out_hbm,
             idx_v, rows_v, lr_v, bias_v, shared_v, sem_g, sem_d, sem_t):
    wid = lax.axis_index("s") * _NC + lax.axis_index("c")
    sid = lax.axis_index("s")
    base = wid * _BPW

    def stage_k(k, carry):
        cid = k * _NS + sid

        @pl.when(cid < _SC_FULL)
        def _full():
            start = cid * _SC_CH
            pltpu.async_copy(table_hbm.at[0, pl.ds(start, _SC_CH)],
                             shared_v.at[pl.ds(start, _SC_CH)], sem_t)

        @pl.when(cid == _SC_FULL)
        def _partial():
            start = cid * _SC_CH
            pltpu.async_copy(table_hbm.at[0, pl.ds(start, _SC_REM)],
                             shared_v.at[pl.ds(start, _SC_REM)], sem_t)

        return carry

    _N_STAGE = (_SC_FULL + _SC_REM // _SC_CH) // _NS + 2
    lax.fori_loop(0, _N_STAGE, stage_k, 0)

    pltpu.sync_copy(xt_hbm.at[:, pl.ds(base, _BPW)], idx_v)
    pltpu.async_copy(dot_hbm.at[pl.ds(base, _BPW)], lr_v, sem_d)
    pltpu.sync_copy(bias_hbm, bias_v)

    def stage_drain(k, carry):
        cid = k * _NS + sid

        @pl.when(cid < _SC_FULL)
        def _full():
            start = cid * _SC_CH
            pltpu.make_async_copy(table_hbm.at[0, pl.ds(start, _SC_CH)],
                                  shared_v.at[pl.ds(start, _SC_CH)], sem_t).wait()

        @pl.when(cid == _SC_FULL)
        def _partial():
            start = cid * _SC_CH
            pltpu.make_async_copy(table_hbm.at[0, pl.ds(start, _SC_REM)],
                                  shared_v.at[pl.ds(start, _SC_REM)], sem_t).wait()

        return carry

    lax.fori_loop(0, _N_STAGE, stage_drain, 0)
    plsc.subcore_barrier()

    def gather_f(f, carry):
        for c in range(_NGC):
            pltpu.async_copy(
                shared_v.at[idx_v.at[f, pl.ds(c * _GC, _GC)]],
                rows_v.at[f, pl.ds(c * _GC, _GC)],
                sem_g,
            )
        return carry

    def drain_f(f, carry):
        for c in range(_NGC):
            pltpu.make_async_copy(
                shared_v.at[idx_v.at[f, pl.ds(c * _GC, _GC)]],
                rows_v.at[f, pl.ds(c * _GC, _GC)],
                sem_g,
            ).wait()
        return carry

    lax.fori_loop(0, _F, gather_f, 0)
    pltpu.make_async_copy(
        dot_hbm.at[pl.ds(base, _BPW)], lr_v, sem_d).wait()
    lax.fori_loop(0, _F, drain_f, 0)

    bias_vec = bias_v[...]

    def lr_g(g, carry):
        acc = lr_v[pl.ds(g * _L, _L)] + bias_vec
        for f in range(_F):
            acc = acc + rows_v[f, pl.ds(g * _L, _L)]
        lr_v[pl.ds(g * _L, _L)] = acc
        return carry

    lax.fori_loop(0, _BPW // _L, lr_g, 0)

    pltpu.sync_copy(lr_v, out_hbm.at[pl.ds(base, _BPW)])


def _params(tc_tiling=False):
    return pltpu.CompilerParams(
        needs_layout_passes=False, use_tc_tiling_on_sc=tc_tiling)


@functools.cache
def _dot_sc():
    return functools.partial(
        pl.kernel,
        out_type=jax.ShapeDtypeStruct((_B,), jnp.float32),
        mesh=plsc.VectorSubcoreMesh(core_axis_name="c", subcore_axis_name="s"),
        scratch_types=[
            pltpu.VMEM((_FD, 2 * _CH), jnp.float32),
            pltpu.VMEM((_BPW,), jnp.float32),
            pltpu.SemaphoreType.DMA,
        ],
        compiler_params=_params(tc_tiling=True),
    )(_dot_body)


@functools.cache
def _lr_sc():
    return functools.partial(
        pl.kernel,
        out_type=jax.ShapeDtypeStruct((_B,), jnp.float32),
        mesh=plsc.VectorSubcoreMesh(core_axis_name="c", subcore_axis_name="s"),
        scratch_types=[
            pltpu.VMEM((_F, _BPW), jnp.int32),
            pltpu.VMEM((_F, _BPW), jnp.float32),
            pltpu.VMEM((_BPW,), jnp.float32),
            pltpu.VMEM((_L,), jnp.float32),
            pltpu.VMEM_SHARED((_V,), jnp.float32),
            pltpu.SemaphoreType.DMA,
            pltpu.SemaphoreType.DMA,
            pltpu.SemaphoreType.DMA,
        ],
        compiler_params=_params(),
    )(_lr_body)


def kernel(X, feature_emb, lr_table, bias):
    Xt = jnp.asarray(X, jnp.int32).T
    fembt = feature_emb.reshape(_B, _FD).T
    bias16 = jnp.broadcast_to(bias.astype(jnp.float32), (_L,))
    dot = _dot_sc()(fembt)
    out = _lr_sc()(Xt, lr_table.T, bias16, dot)
    return out.reshape(_B, 1)


# speedup vs baseline: 3.5348x; 1.0003x over previous
import functools

import jax
import jax.numpy as jnp
from jax import lax
from jax.experimental import pallas as pl
from jax.experimental.pallas import tpu as pltpu
from jax.experimental.pallas import tpu_sc as plsc

_B, _F, _D, _V = 16384, 26, 16, 1000000
_NC, _NS, _L = 2, 16, 16
_NW = _NC * _NS
_BPW = _B // _NW
_GC = 128
_NGC = _BPW // _GC
_CH = 128
_NCH = _BPW // _CH
_FD = _F * _D


def _dot_body(fembt_hbm, out_hbm, dbuf_v, dot_v, sem_d):
    wid = lax.axis_index("s") * _NC + lax.axis_index("c")
    base = wid * _BPW

    pltpu.async_copy(
        fembt_hbm.at[:, pl.ds(base, _CH)],
        dbuf_v.at[:, pl.ds(0, _CH)], sem_d)

    def chunk_t(t, carry):
        buf = t % 2
        pltpu.make_async_copy(
            fembt_hbm.at[:, pl.ds(base, _CH)],
            dbuf_v.at[:, pl.ds(0, _CH)], sem_d).wait()

        @pl.when(t + 1 < _NCH)
        def _prefetch():
            pltpu.async_copy(
                fembt_hbm.at[:, pl.ds(base + (t + 1) * _CH, _CH)],
                dbuf_v.at[:, pl.ds(((t + 1) % 2) * _CH, _CH)],
                sem_d)

        def grp(g, inner):
            off = buf * _CH + g * _L
            zero = jnp.zeros((_L,), jnp.float32)

            def facc(f, carry):
                acc_sq = carry[0]
                acc_d = list(carry[1:])
                for d in range(_D):
                    v = dbuf_v[f * _D + d, pl.ds(off, _L)]
                    acc_sq = acc_sq + v * v
                    acc_d[d] = acc_d[d] + v
                return (acc_sq, *acc_d)

            res = lax.fori_loop(0, _F, facc, (zero,) * (_D + 1))
            r = -res[0]
            for d in range(_D):
                r = r + res[1 + d] * res[1 + d]
            gg = t * (_CH // _L) + g
            dot_v[pl.ds(gg * _L, _L)] = 0.5 * r
            return inner

        lax.fori_loop(0, _CH // _L, grp, 0)
        return carry

    lax.fori_loop(0, _NCH, chunk_t, 0)

    pltpu.sync_copy(dot_v, out_hbm.at[pl.ds(base, _BPW)])


_SC_CH = 4096
_SC_FULL = _V // _SC_CH
_SC_REM = _V - _SC_FULL * _SC_CH


def _lr_body(xt_hbm, table_hbm, bias_hbm, dot_hbm, out_hbm,
             idx_v, rows_v, lr_v, bias_v, shared_v, sem_g, sem_d, sem_t):
    wid = lax.axis_index("s") * _NC + lax.axis_index("c")
    sid = lax.axis_index("s")
    base = wid * _BPW

    def stage_k(k, carry):
        cid = k * _NS + sid

        @pl.when(cid < _SC_FULL)
        def _full():
            start = cid * _SC_CH
            pltpu.async_copy(table_hbm.at[0, pl.ds(start, _SC_CH)],
                             shared_v.at[pl.ds(start, _SC_CH)], sem_t)

        @pl.when(cid == _SC_FULL)
        def _partial():
            start = cid * _SC_CH
            pltpu.async_copy(table_hbm.at[0, pl.ds(start, _SC_REM)],
                             shared_v.at[pl.ds(start, _SC_REM)], sem_t)

        return carry

    _N_STAGE = (_SC_FULL + _SC_REM // _SC_CH) // _NS + 2
    lax.fori_loop(0, _N_STAGE, stage_k, 0)

    pltpu.sync_copy(xt_hbm.at[:, pl.ds(base, _BPW)], idx_v)
    pltpu.async_copy(dot_hbm.at[pl.ds(base, _BPW)], lr_v, sem_d)
    pltpu.sync_copy(bias_hbm, bias_v)

    def stage_drain(k, carry):
        cid = k * _NS + sid

        @pl.when(cid < _SC_FULL)
        def _full():
            start = cid * _SC_CH
            pltpu.make_async_copy(table_hbm.at[0, pl.ds(start, _SC_CH)],
                                  shared_v.at[pl.ds(start, _SC_CH)], sem_t).wait()

        @pl.when(cid == _SC_FULL)
        def _partial():
            start = cid * _SC_CH
            pltpu.make_async_copy(table_hbm.at[0, pl.ds(start, _SC_REM)],
                                  shared_v.at[pl.ds(start, _SC_REM)], sem_t).wait()

        return carry

    lax.fori_loop(0, _N_STAGE, stage_drain, 0)
    plsc.subcore_barrier()

    def gather_f(f, carry):
        for c in range(_NGC):
            pltpu.async_copy(
                shared_v.at[idx_v.at[f, pl.ds(c * _GC, _GC)]],
                rows_v.at[f, pl.ds(c * _GC, _GC)],
                sem_g,
            )
        return carry

    def drain_f(f, carry):
        for c in range(_NGC):
            pltpu.make_async_copy(
                shared_v.at[idx_v.at[f, pl.ds(c * _GC, _GC)]],
                rows_v.at[f, pl.ds(c * _GC, _GC)],
                sem_g,
            ).wait()
        return carry

    lax.fori_loop(0, _F, gather_f, 0)
    pltpu.make_async_copy(
        dot_hbm.at[pl.ds(base, _BPW)], lr_v, sem_d).wait()
    lax.fori_loop(0, _F, drain_f, 0)

    bias_vec = bias_v[...]

    def lr_g(g, carry):
        acc = lr_v[pl.ds(g * _L, _L)] + bias_vec
        for f in range(_F):
            acc = acc + rows_v[f, pl.ds(g * _L, _L)]
        lr_v[pl.ds(g * _L, _L)] = acc
        return carry

    lax.fori_loop(0, _BPW // _L, lr_g, 0)

    pltpu.sync_copy(lr_v, out_hbm.at[pl.ds(base, _BPW)])


def _params(tc_tiling=False):
    return pltpu.CompilerParams(
        needs_layout_passes=False, use_tc_tiling_on_sc=tc_tiling)


@functools.cache
def _dot_sc():
    return functools.partial(
        pl.kernel,
        out_type=jax.ShapeDtypeStruct((_B,), jnp.float32),
        mesh=plsc.VectorSubcoreMesh(core_axis_name="c", subcore_axis_name="s"),
        scratch_types=[
            pltpu.VMEM((_FD, 2 * _CH), jnp.float32),
            pltpu.VMEM((_BPW,), jnp.float32),
            pltpu.SemaphoreType.DMA,
        ],
        compiler_params=_params(tc_tiling=True),
    )(_dot_body)


@functools.cache
def _lr_sc():
    return functools.partial(
        pl.kernel,
        out_type=jax.ShapeDtypeStruct((_B,), jnp.float32),
        mesh=plsc.VectorSubcoreMesh(core_axis_name="c", subcore_axis_name="s"),
        scratch_types=[
            pltpu.VMEM((_F, _BPW), jnp.int32),
            pltpu.VMEM((_F, _BPW), jnp.float32),
            pltpu.VMEM((_BPW,), jnp.float32),
            pltpu.VMEM((_L,), jnp.float32),
            pltpu.VMEM_SHARED((_V,), jnp.float32),
            pltpu.SemaphoreType.DMA,
            pltpu.SemaphoreType.DMA,
            pltpu.SemaphoreType.DMA,
        ],
        compiler_params=_params(),
    )(_lr_body)


def kernel(X, feature_emb, lr_table, bias):
    Xt = jnp.asarray(X, jnp.int32).T
    fembt = feature_emb.reshape(_B, _FD).T
    bias16 = jnp.broadcast_to(bias.astype(jnp.float32), (_L,))
    dot = _dot_sc()(fembt)
    out = _lr_sc()(Xt, jnp.reshape(lr_table, (1, _V)), bias16, dot)
    return out.reshape(_B, 1)


# speedup vs baseline: 3.5450x; 1.0029x over previous
import functools

import jax
import jax.numpy as jnp
from jax import lax
from jax.experimental import pallas as pl
from jax.experimental.pallas import tpu as pltpu
from jax.experimental.pallas import tpu_sc as plsc

_B, _F, _D, _V = 16384, 26, 16, 1000000
_NC, _NS, _L = 2, 16, 16
_NW = _NC * _NS
_BPW = _B // _NW
_GC = 128
_NGC = _BPW // _GC
_CH = 128
_NCH = _BPW // _CH
_FD = _F * _D


def _dot_body(fembt_hbm, xt_hbm, out_hbm, xt_out_hbm, dbuf_v, dot_v, ibuf_v, sem_d):
    wid = lax.axis_index("s") * _NC + lax.axis_index("c")
    base = wid * _BPW

    pltpu.sync_copy(xt_hbm.at[:, pl.ds(base, _BPW)], ibuf_v)
    pltpu.sync_copy(ibuf_v, xt_out_hbm.at[:, pl.ds(base, _BPW)])

    pltpu.async_copy(
        fembt_hbm.at[:, pl.ds(base, _CH)],
        dbuf_v.at[:, pl.ds(0, _CH)], sem_d)

    def chunk_t(t, carry):
        buf = t % 2
        pltpu.make_async_copy(
            fembt_hbm.at[:, pl.ds(base, _CH)],
            dbuf_v.at[:, pl.ds(0, _CH)], sem_d).wait()

        @pl.when(t + 1 < _NCH)
        def _prefetch():
            pltpu.async_copy(
                fembt_hbm.at[:, pl.ds(base + (t + 1) * _CH, _CH)],
                dbuf_v.at[:, pl.ds(((t + 1) % 2) * _CH, _CH)],
                sem_d)

        def grp(g, inner):
            off = buf * _CH + g * _L
            zero = jnp.zeros((_L,), jnp.float32)

            def facc(f, carry):
                acc_sq = carry[0]
                acc_d = list(carry[1:])
                for d in range(_D):
                    v = dbuf_v[f * _D + d, pl.ds(off, _L)]
                    acc_sq = acc_sq + v * v
                    acc_d[d] = acc_d[d] + v
                return (acc_sq, *acc_d)

            res = lax.fori_loop(0, _F, facc, (zero,) * (_D + 1))
            r = -res[0]
            for d in range(_D):
                r = r + res[1 + d] * res[1 + d]
            gg = t * (_CH // _L) + g
            dot_v[pl.ds(gg * _L, _L)] = 0.5 * r
            return inner

        lax.fori_loop(0, _CH // _L, grp, 0)
        return carry

    lax.fori_loop(0, _NCH, chunk_t, 0)

    pltpu.sync_copy(dot_v, out_hbm.at[pl.ds(base, _BPW)])


_SC_CH = 4096
_SC_FULL = _V // _SC_CH
_SC_REM = _V - _SC_FULL * _SC_CH


def _lr_body(xt_hbm, table_hbm, bias_hbm, dot_hbm, out_hbm,
             idx_v, rows_v, lr_v, bias_v, shared_v, sem_g, sem_d, sem_t):
    wid = lax.axis_index("s") * _NC + lax.axis_index("c")
    sid = lax.axis_index("s")
    base = wid * _BPW

    def stage_k(k, carry):
        cid = k * _NS + sid

        @pl.when(cid < _SC_FULL)
        def _full():
            start = cid * _SC_CH
            pltpu.async_copy(table_hbm.at[0, pl.ds(start, _SC_CH)],
                             shared_v.at[pl.ds(start, _SC_CH)], sem_t)

        @pl.when(cid == _SC_FULL)
        def _partial():
            start = cid * _SC_CH
            pltpu.async_copy(table_hbm.at[0, pl.ds(start, _SC_REM)],
                             shared_v.at[pl.ds(start, _SC_REM)], sem_t)

        return carry

    _N_STAGE = (_SC_FULL + _SC_REM // _SC_CH) // _NS + 2
    lax.fori_loop(0, _N_STAGE, stage_k, 0)

    pltpu.sync_copy(xt_hbm.at[:, pl.ds(base, _BPW)], idx_v)
    pltpu.async_copy(dot_hbm.at[pl.ds(base, _BPW)], lr_v, sem_d)
    pltpu.sync_copy(bias_hbm, bias_v)

    def stage_drain(k, carry):
        cid = k * _NS + sid

        @pl.when(cid < _SC_FULL)
        def _full():
            start = cid * _SC_CH
            pltpu.make_async_copy(table_hbm.at[0, pl.ds(start, _SC_CH)],
                                  shared_v.at[pl.ds(start, _SC_CH)], sem_t).wait()

        @pl.when(cid == _SC_FULL)
        def _partial():
            start = cid * _SC_CH
            pltpu.make_async_copy(table_hbm.at[0, pl.ds(start, _SC_REM)],
                                  shared_v.at[pl.ds(start, _SC_REM)], sem_t).wait()

        return carry

    lax.fori_loop(0, _N_STAGE, stage_drain, 0)
    plsc.subcore_barrier()

    def gather_f(f, carry):
        for c in range(_NGC):
            pltpu.async_copy(
                shared_v.at[idx_v.at[f, pl.ds(c * _GC, _GC)]],
                rows_v.at[f, pl.ds(c * _GC, _GC)],
                sem_g,
            )
        return carry

    def drain_f(f, carry):
        for c in range(_NGC):
            pltpu.make_async_copy(
                shared_v.at[idx_v.at[f, pl.ds(c * _GC, _GC)]],
                rows_v.at[f, pl.ds(c * _GC, _GC)],
                sem_g,
            ).wait()
        return carry

    lax.fori_loop(0, _F, gather_f, 0)
    pltpu.make_async_copy(
        dot_hbm.at[pl.ds(base, _BPW)], lr_v, sem_d).wait()
    lax.fori_loop(0, _F, drain_f, 0)

    bias_vec = bias_v[...]

    def lr_g(g, carry):
        acc = lr_v[pl.ds(g * _L, _L)] + bias_vec
        for f in range(_F):
            acc = acc + rows_v[f, pl.ds(g * _L, _L)]
        lr_v[pl.ds(g * _L, _L)] = acc
        return carry

    lax.fori_loop(0, _BPW // _L, lr_g, 0)

    pltpu.sync_copy(lr_v, out_hbm.at[pl.ds(base, _BPW)])


def _params(tc_tiling=False):
    return pltpu.CompilerParams(
        needs_layout_passes=False, use_tc_tiling_on_sc=tc_tiling)


@functools.cache
def _dot_sc():
    return functools.partial(
        pl.kernel,
        out_type=(jax.ShapeDtypeStruct((_B,), jnp.float32),
                  jax.ShapeDtypeStruct((_F, _B), jnp.int32)),
        mesh=plsc.VectorSubcoreMesh(core_axis_name="c", subcore_axis_name="s"),
        scratch_types=[
            pltpu.VMEM((_FD, 2 * _CH), jnp.float32),
            pltpu.VMEM((_BPW,), jnp.float32),
            pltpu.VMEM((_F, _BPW), jnp.int32),
            pltpu.SemaphoreType.DMA,
        ],
        compiler_params=_params(tc_tiling=True),
    )(_dot_body)


@functools.cache
def _lr_sc():
    return functools.partial(
        pl.kernel,
        out_type=jax.ShapeDtypeStruct((_B,), jnp.float32),
        mesh=plsc.VectorSubcoreMesh(core_axis_name="c", subcore_axis_name="s"),
        scratch_types=[
            pltpu.VMEM((_F, _BPW), jnp.int32),
            pltpu.VMEM((_F, _BPW), jnp.float32),
            pltpu.VMEM((_BPW,), jnp.float32),
            pltpu.VMEM((_L,), jnp.float32),
            pltpu.VMEM_SHARED((_V,), jnp.float32),
            pltpu.SemaphoreType.DMA,
            pltpu.SemaphoreType.DMA,
            pltpu.SemaphoreType.DMA,
        ],
        compiler_params=_params(),
    )(_lr_body)


def kernel(X, feature_emb, lr_table, bias):
    Xt = jnp.asarray(X, jnp.int32).T
    fembt = feature_emb.reshape(_B, _FD).T
    bias16 = jnp.broadcast_to(bias.astype(jnp.float32), (_L,))
    dot, xt_lin = _dot_sc()(fembt, Xt)
    out = _lr_sc()(xt_lin, jnp.reshape(lr_table, (1, _V)), bias16, dot)
    return out.reshape(_B, 1)
